# baseline (speedup 1.0000x reference)
import functools

import jax
import jax.numpy as jnp
from jax.experimental import pallas as pl
from jax.experimental.pallas import tpu as pltpu
from jax.experimental.pallas import tpu_sc as plsc

_D_MODEL = 4096
_THRESHOLD = 0.5
_CAPACITY_FRACTION = 0.5
_ROWS = 512

_LANES = 16


def _matvec_body(h_ref, w_ref, b_ref, s_ref):
    logits = jnp.dot(h_ref[...], w_ref[...], preferred_element_type=jnp.float32)
    s_ref[...] = jax.nn.sigmoid(logits + b_ref[0, 0])


def _sc_mask_call(scores2d, exited2d, B, T, k_cap):
    mesh = plsc.VectorSubcoreMesh(core_axis_name="c", subcore_axis_name="s")
    n_chunks = T // _LANES

    @functools.partial(
        pl.kernel,
        mesh=mesh,
        out_type=jax.ShapeDtypeStruct((B, T), jnp.int32),
        scratch_types=[
            pltpu.VMEM((T,), jnp.float32),
            pltpu.VMEM((T,), jnp.int32),
            pltpu.VMEM((T,), jnp.int32),
            pltpu.VMEM((_LANES,), jnp.int32),
            pltpu.VMEM((T,), jnp.int32),
        ],
    )
    def run(s_hbm, e_hbm, out_hbm, s_v, e_v, t_v, m_v, o_v):
        wid = jax.lax.axis_index("s") * 2 + jax.lax.axis_index("c")

        @pl.when(wid < B)
        def _row():
            zeros16 = jnp.zeros((_LANES,), jnp.int32)
            ones16 = jnp.full((_LANES,), 1, jnp.int32)

            pltpu.sync_copy(s_hbm.at[wid], s_v)
            pltpu.sync_copy(e_hbm.at[wid], e_v)

            def keys_at(i):
                return jax.lax.bitcast_convert_type(
                    s_v[pl.ds(i * _LANES, _LANES)], jnp.int32)

            def count_ge(cand):
                def cb(i, acc):
                    return acc + jnp.where(keys_at(i) >= cand, ones16, zeros16)
                acc = jax.lax.fori_loop(0, n_chunks, cb, zeros16)
                tot = acc[0]
                for l in range(1, _LANES):
                    tot = tot + acc[l]
                return tot

            def tau_body(bit, tau):
                cand = tau | (jnp.int32(1) << (jnp.int32(30) - bit))
                return jnp.where(count_ge(cand) >= k_cap, cand, tau)

            tau = jax.lax.fori_loop(0, 31, tau_body, jnp.int32(0))

            n_ge = count_ge(tau)
            n_gt = count_ge(tau + 1)
            need = k_cap - n_gt

            m_v[...] = jnp.full((_LANES,), T, jnp.int32)

            @pl.when(n_ge != k_cap)
            def _partial_ties():
                def tie_pass(i, c):
                    t_v[pl.ds(i * _LANES, _LANES)] = jnp.where(
                        keys_at(i) == tau, ones16, zeros16)
                    return c
                jax.lax.fori_loop(0, n_chunks, tie_pass, 0)

                def tie_chunk(c, carry):
                    cnt, m = carry
                    t16 = t_v[pl.ds(c * _LANES, _LANES)]
                    for l in range(_LANES):
                        cnt = cnt + t16[l]
                        m = jnp.where(
                            (t16[l] > 0) & (m < 0) & (cnt == need),
                            c * _LANES + l, m)
                    return (cnt, m)

                _, m = jax.lax.fori_loop(
                    0, n_chunks, tie_chunk, (jnp.int32(0), jnp.int32(-1)))
                m_v[...] = jnp.full((_LANES,), 1, jnp.int32) * m

            m = m_v[...][0]

            iota16 = jax.lax.iota(jnp.int32, _LANES)

            def fin(i, c):
                k16 = keys_at(i)
                gidx = i * _LANES + iota16
                sel = (k16 > tau) | ((k16 == tau) & (gidx <= m))
                keep = (sel & (s_v[pl.ds(i * _LANES, _LANES)] > _THRESHOLD)
                        & (e_v[pl.ds(i * _LANES, _LANES)] == 0))
                o_v[pl.ds(i * _LANES, _LANES)] = jnp.where(
                    keep, ones16, zeros16)
                return c
            jax.lax.fori_loop(0, n_chunks, fin, 0)

            pltpu.sync_copy(o_v, out_hbm.at[wid])

    return run(scores2d, exited2d)


def kernel(h, exited_so_far, W, b):
    B, T, D = h.shape
    k_cap = max(1, min(T, int(_CAPACITY_FRACTION * T + 0.5)))

    h_flat = h.reshape(B * T, D)
    w_col = W.reshape(D, 1)
    b2 = b.reshape(1, 1)

    n_chunks = (B * T) // _ROWS
    scores_flat = pl.pallas_call(
        _matvec_body,
        grid=(n_chunks,),
        in_specs=[
            pl.BlockSpec((_ROWS, D), lambda i: (i, 0)),
            pl.BlockSpec((D, 1), lambda i: (0, 0)),
            pl.BlockSpec(memory_space=pltpu.SMEM),
        ],
        out_specs=pl.BlockSpec((_ROWS, 1), lambda i: (i, 0)),
        out_shape=jax.ShapeDtypeStruct((B * T, 1), jnp.float32),
    )(h_flat, w_col, b2)

    scores2d = scores_flat.reshape(B, T)
    exited2d = exited_so_far.reshape(B, T).astype(jnp.int32)

    mask2d = _sc_mask_call(scores2d, exited2d, B, T, k_cap)

    scores = scores_flat.reshape(B, T, 1)
    exit_mask = mask2d.astype(jnp.bool_).reshape(B, T, 1)
    return (scores, exit_mask)


# speedup vs baseline: 1.2030x; 1.2030x over previous
import functools

import jax
import jax.numpy as jnp
from jax.experimental import pallas as pl
from jax.experimental.pallas import tpu as pltpu
from jax.experimental.pallas import tpu_sc as plsc

_D_MODEL = 4096
_THRESHOLD = 0.5
_CAPACITY_FRACTION = 0.5
_ROWS = 512

_LANES = 16


def _matvec_body(h_ref, w_ref, b_ref, s_ref):
    logits = jnp.dot(h_ref[...], w_ref[...], preferred_element_type=jnp.float32)
    s_ref[...] = jax.nn.sigmoid(logits + b_ref[0, 0])


def _sc_mask_call(scores2d, exited2d, B, T, k_cap):
    mesh = plsc.VectorSubcoreMesh(core_axis_name="c", subcore_axis_name="s")
    n_chunks = T // _LANES

    @functools.partial(
        pl.kernel,
        mesh=mesh,
        out_type=jax.ShapeDtypeStruct((B, T), jnp.int32),
        scratch_types=[
            pltpu.VMEM((T,), jnp.float32),
            pltpu.VMEM((T,), jnp.int32),
            pltpu.VMEM((T,), jnp.int32),
            pltpu.VMEM((_LANES,), jnp.int32),
            pltpu.VMEM((T,), jnp.int32),
        ],
    )
    def run(s_hbm, e_hbm, out_hbm, s_v, e_v, t_v, m_v, o_v):
        wid = jax.lax.axis_index("s") * 2 + jax.lax.axis_index("c")

        @pl.when(wid < B)
        def _row():
            zeros16 = jnp.zeros((_LANES,), jnp.int32)
            ones16 = jnp.full((_LANES,), 1, jnp.int32)

            pltpu.sync_copy(s_hbm.at[wid], s_v)
            pltpu.sync_copy(e_hbm.at[wid], e_v)

            def keys_at(i):
                return jax.lax.bitcast_convert_type(
                    s_v[pl.ds(i * _LANES, _LANES)], jnp.int32)

            def count_ge(cand):
                def cb(i, accs):
                    return tuple(
                        accs[u] + jnp.where(keys_at(i * 4 + u) >= cand,
                                            ones16, zeros16)
                        for u in range(4))
                accs = jax.lax.fori_loop(
                    0, n_chunks // 4, cb, (zeros16,) * 4)
                acc = (accs[0] + accs[1]) + (accs[2] + accs[3])
                lanes = [acc[l] for l in range(_LANES)]
                while len(lanes) > 1:
                    lanes = [lanes[i] + lanes[i + 1]
                             for i in range(0, len(lanes), 2)]
                return lanes[0]

            def tau_body(bit, tau):
                cand = tau | (jnp.int32(1) << (jnp.int32(30) - bit))
                return jnp.where(count_ge(cand) >= k_cap, cand, tau)

            tau = jax.lax.fori_loop(0, 31, tau_body, jnp.int32(0))

            n_ge = count_ge(tau)
            n_gt = count_ge(tau + 1)
            need = k_cap - n_gt

            m_v[...] = jnp.full((_LANES,), T, jnp.int32)

            @pl.when(n_ge != k_cap)
            def _partial_ties():
                def tie_pass(i, c):
                    t_v[pl.ds(i * _LANES, _LANES)] = jnp.where(
                        keys_at(i) == tau, ones16, zeros16)
                    return c
                jax.lax.fori_loop(0, n_chunks, tie_pass, 0)

                def tie_chunk(c, carry):
                    cnt, m = carry
                    t16 = t_v[pl.ds(c * _LANES, _LANES)]
                    for l in range(_LANES):
                        cnt = cnt + t16[l]
                        m = jnp.where(
                            (t16[l] > 0) & (m < 0) & (cnt == need),
                            c * _LANES + l, m)
                    return (cnt, m)

                _, m = jax.lax.fori_loop(
                    0, n_chunks, tie_chunk, (jnp.int32(0), jnp.int32(-1)))
                m_v[...] = jnp.full((_LANES,), 1, jnp.int32) * m

            m = m_v[...][0]

            iota16 = jax.lax.iota(jnp.int32, _LANES)

            def fin(i, c):
                k16 = keys_at(i)
                gidx = i * _LANES + iota16
                sel = (k16 > tau) | ((k16 == tau) & (gidx <= m))
                keep = (sel & (s_v[pl.ds(i * _LANES, _LANES)] > _THRESHOLD)
                        & (e_v[pl.ds(i * _LANES, _LANES)] == 0))
                o_v[pl.ds(i * _LANES, _LANES)] = jnp.where(
                    keep, ones16, zeros16)
                return c
            jax.lax.fori_loop(0, n_chunks, fin, 0)

            pltpu.sync_copy(o_v, out_hbm.at[wid])

    return run(scores2d, exited2d)


def kernel(h, exited_so_far, W, b):
    B, T, D = h.shape
    k_cap = max(1, min(T, int(_CAPACITY_FRACTION * T + 0.5)))

    h_flat = h.reshape(B * T, D)
    w_col = W.reshape(D, 1)
    b2 = b.reshape(1, 1)

    n_chunks = (B * T) // _ROWS
    scores_flat = pl.pallas_call(
        _matvec_body,
        grid=(n_chunks,),
        in_specs=[
            pl.BlockSpec((_ROWS, D), lambda i: (i, 0)),
            pl.BlockSpec((D, 1), lambda i: (0, 0)),
            pl.BlockSpec(memory_space=pltpu.SMEM),
        ],
        out_specs=pl.BlockSpec((_ROWS, 1), lambda i: (i, 0)),
        out_shape=jax.ShapeDtypeStruct((B * T, 1), jnp.float32),
    )(h_flat, w_col, b2)

    scores2d = scores_flat.reshape(B, T)
    exited2d = exited_so_far.reshape(B, T).astype(jnp.int32)

    mask2d = _sc_mask_call(scores2d, exited2d, B, T, k_cap)

    scores = scores_flat.reshape(B, T, 1)
    exit_mask = mask2d.astype(jnp.bool_).reshape(B, T, 1)
    return (scores, exit_mask)
